# baseline (speedup 1.0000x reference)
import functools

import jax
import jax.numpy as jnp
from jax.experimental import pallas as pl
from jax.experimental.pallas import tpu as pltpu

_LANES = 128


def _chamfer_body(x1_ref, x2_ref, out_ref, *, nb_size, mb_size, m_total, inv_count):
    b = pl.program_id(0)
    nb = pl.program_id(1)
    last_b = pl.num_programs(0) - 1
    last_nb = pl.num_programs(1) - 1

    x1 = x1_ref[0]
    x1x = x1[:, 0:1]
    x1y = x1[:, 1:2]
    x1z = x1[:, 2:3]
    n1 = x1x * x1x + x1y * x1y + x1z * x1z

    def chunk(i, carry):
        best_dd, best_true = carry
        xs = x2_ref[0, :, pl.ds(i * mb_size, mb_size)]
        xsx = xs[0:1, :]
        xsy = xs[1:2, :]
        xsz = xs[2:3, :]
        n2 = xsx * xsx + xsy * xsy + xsz * xsz
        dot = jnp.dot(x1, xs, preferred_element_type=jnp.float32)
        dd = n1 + n2 - 2.0 * dot
        dx = x1x - xsx
        dy = x1y - xsy
        dz = x1z - xsz
        td = dx * dx + dy * dy + dz * dz
        for j in range(mb_size // _LANES):
            sl = slice(j * _LANES, (j + 1) * _LANES)
            dd_j = dd[:, sl]
            td_j = td[:, sl]
            upd = dd_j < best_dd
            best_dd = jnp.where(upd, dd_j, best_dd)
            best_true = jnp.where(upd, td_j, best_true)
        return best_dd, best_true

    inf = jnp.full((nb_size, _LANES), jnp.inf, dtype=jnp.float32)
    best_dd, best_true = jax.lax.fori_loop(
        0, m_total // mb_size, chunk, (inf, inf))
    rowmin = jnp.min(best_dd, axis=1, keepdims=True)
    sel = jnp.where(best_dd == rowmin, best_true, jnp.inf)
    s = jnp.sum(jnp.min(sel, axis=1))

    is_first = jnp.logical_and(b == 0, nb == 0)
    prev = jnp.where(is_first, jnp.float32(0.0), out_ref[0, 0])
    total = prev + s
    is_last = jnp.logical_and(b == last_b, nb == last_nb)
    out_ref[0, 0] = jnp.where(is_last, total * inv_count, total)


def kernel(xyz1, xyz2):
    B, N, _ = xyz1.shape
    M = xyz2.shape[1]
    nb_size = min(256, N)
    mb_size = min(1024, M)

    x2t = jnp.transpose(xyz2, (0, 2, 1))
    body = functools.partial(
        _chamfer_body,
        nb_size=nb_size,
        mb_size=mb_size,
        m_total=M,
        inv_count=1.0 / (B * N),
    )
    out = pl.pallas_call(
        body,
        grid=(B, N // nb_size),
        in_specs=[
            pl.BlockSpec((1, nb_size, 3), lambda b, nb: (b, nb, 0)),
            pl.BlockSpec((1, 3, M), lambda b, nb: (b, 0, 0)),
        ],
        out_specs=pl.BlockSpec(memory_space=pltpu.SMEM),
        out_shape=jax.ShapeDtypeStruct((1, 1), jnp.float32),
    )(xyz1, x2t)
    return out[0, 0]
